# Initial kernel scaffold; baseline (speedup 1.0000x reference)
import jax
import jax.numpy as jnp
from jax.experimental import pallas as pl


def kernel(hidden_states, Wg, W1, W3, W2):
    raise NotImplementedError("write your pallas kernel here")


# speedup vs baseline: 2.1451x; 2.1451x over previous
import functools

import jax
import jax.numpy as jnp
from jax.experimental import pallas as pl
from jax.experimental.pallas import tpu as pltpu
from jax.experimental.pallas import tpu_sc as plsc

TOPK = 2
MB = 512
FB = 512
SC_WIN = 128


def _iota(shape, dim):
    return jax.lax.broadcasted_iota(jnp.int32, shape, dim)


def _col_to_qc(xcol, Q, C, precision=None):
    N = xcol.shape[0]
    M = (_iota((N, C), 0) % C == _iota((N, C), 1)).astype(jnp.float32)
    XM = xcol * M
    L = (_iota((Q, N), 1) // C == _iota((Q, N), 0)).astype(jnp.float32)
    return jax.lax.dot_general(L, XM, (((1,), (0,)), ((), ())),
                               precision=precision,
                               preferred_element_type=jnp.float32)


def _router_body(x_ref, wg_ref, w0_ref, w1_ref, dest_ref, blk_ref):
    S, E = x_ref.shape[0], wg_ref.shape[1]
    P = 2 * S
    Q = S // 128
    NB = (2 * S + (E - 1) * MB) // MB
    HI = jax.lax.Precision.HIGHEST

    x = x_ref[...]

    logits = jax.lax.dot_general(x, wg_ref[...], (((1,), (0,)), ((), ())),
                                 preferred_element_type=jnp.float32)
    m = jnp.max(logits, axis=1, keepdims=True)
    p = jnp.exp(logits - m)
    p = p / jnp.sum(p, axis=1, keepdims=True)
    lane = _iota((S, E), 1)
    i0 = jnp.min(jnp.where(logits == m, lane, E), axis=1, keepdims=True)
    l2 = jnp.where(lane == i0, -jnp.inf, logits)
    m1 = jnp.max(l2, axis=1, keepdims=True)
    i1 = jnp.min(jnp.where(l2 == m1, lane, E), axis=1, keepdims=True)
    oh0 = (lane == i0)
    oh1 = (lane == i1)
    w0_ref[...] = jnp.sum(jnp.where(oh0, p, 0.0), axis=1, keepdims=True)
    w1_ref[...] = jnp.sum(jnp.where(oh1, p, 0.0), axis=1, keepdims=True)

    i0qc = _col_to_qc(i0.astype(jnp.float32), Q, 128)
    i1qc = _col_to_qc(i1.astype(jnp.float32), Q, 128)
    ep = jnp.concatenate([i0qc, i1qc], axis=0)

    R = E * 2 * Q
    T = (_iota((R, 2 * Q), 0) % (2 * Q) == _iota((R, 2 * Q), 1))
    ep_t = jax.lax.dot_general(T.astype(jnp.float32), ep,
                               (((1,), (0,)), ((), ())),
                               preferred_element_type=jnp.float32)
    e_of_r = (_iota((R, 128), 0) // (2 * Q)).astype(jnp.float32)
    A2 = (ep_t == e_of_r).astype(jnp.float32)

    U128 = (_iota((128, 128), 0) <= _iota((128, 128), 1)).astype(jnp.float32)
    inrow = jax.lax.dot_general(A2, U128, (((1,), (0,)), ((), ())),
                                preferred_element_type=jnp.float32)
    rowtot = inrow[:, 127:128]

    rr = _iota((R, R), 0)
    cc = _iota((R, R), 1)
    same_e = (rr // (2 * Q)) == (cc // (2 * Q))
    prev_row = same_e & ((cc % (2 * Q)) < (rr % (2 * Q)))
    Pexcl = jax.lax.dot_general(prev_row.astype(jnp.float32), rowtot,
                                (((1,), (0,)), ((), ())),
                                preferred_element_type=jnp.float32)
    counts_pg = jax.lax.dot_general(same_e.astype(jnp.float32), rowtot,
                                    (((1,), (0,)), ((), ())),
                                    preferred_element_type=jnp.float32)
    pc_pg = jnp.floor((counts_pg + (MB - 1)) / MB) * MB
    prev_e = ((cc // (2 * Q)) < (rr // (2 * Q))).astype(jnp.float32)
    base_pg = jax.lax.dot_general(prev_e, pc_pg, (((1,), (0,)), ((), ())),
                                  preferred_element_type=jnp.float32)
    base_pg = base_pg / (2 * Q)

    cum = inrow + Pexcl
    dest_pg = (base_pg + cum - 1.0) * A2

    G = (_iota((2 * Q, R), 1) % (2 * Q) == _iota((2 * Q, R), 0))
    dest32 = jax.lax.dot_general(G.astype(jnp.float32), dest_pg,
                                 (((1,), (0,)), ((), ())),
                                 precision=HI, preferred_element_type=jnp.float32)
    dest_ref[...] = dest32.astype(jnp.int32)

    sel = (_iota((R, E), 0) == _iota((R, E), 1) * (2 * Q)).astype(jnp.float32)
    base8 = jax.lax.dot_general(sel, base_pg, (((0,), (0,)), ((), ())),
                                preferred_element_type=jnp.float32)
    pc8 = jax.lax.dot_general(sel, pc_pg, (((0,), (0,)), ((), ())),
                              preferred_element_type=jnp.float32)
    border = (_iota((E, NB), 1) * MB).astype(jnp.float32)
    owns = ((base8 <= border) & (border < base8 + pc8)).astype(jnp.float32)
    e_col = _iota((E, 1), 0).astype(jnp.float32)
    blk_e = jnp.sum(owns * e_col, axis=0, keepdims=True)
    blk_v = jnp.sum(owns, axis=0, keepdims=True)
    blk_ref[...] = jnp.where(blk_v > 0.5, blk_e, jnp.float32(E)).astype(jnp.int32)


def _run_router(x, Wg):
    S, d = x.shape
    E = Wg.shape[1]
    NB = (2 * S + (E - 1) * MB) // MB
    return pl.pallas_call(
        _router_body,
        out_shape=(
            jax.ShapeDtypeStruct((S, 1), jnp.float32),
            jax.ShapeDtypeStruct((S, 1), jnp.float32),
            jax.ShapeDtypeStruct((2 * (S // 128), 128), jnp.int32),
            jax.ShapeDtypeStruct((1, NB), jnp.int32),
        ),
    )(x, Wg)


_SC_NC = 2
_SC_NS = 16
_SC_NW = _SC_NC * _SC_NS
_SC_CHUNK = 32


def _sc_scatter(x, idx2d, P):
    S, d = x.shape
    nrows, ck = idx2d.shape
    n = nrows * ck
    per_w = n // _SC_NW
    nch = per_w // ck
    mesh = plsc.VectorSubcoreMesh(core_axis_name="c", subcore_axis_name="s")

    @functools.partial(
        pl.kernel, mesh=mesh,
        out_type=jax.ShapeDtypeStruct((P, d), x.dtype),
        scratch_types=[
            pltpu.VMEM((ck,), jnp.int32),
            pltpu.VMEM((ck, d), x.dtype),
            pltpu.SemaphoreType.DMA,
        ],
    )
    def kern(x_hbm, i_hbm, o_hbm, idx_v, data_v, sem):
        wid = jax.lax.axis_index("s") * _SC_NC + jax.lax.axis_index("c")

        @pl.loop(0, nch)
        def _(c):
            j0 = wid * per_w + c * ck
            src = jax.lax.rem(j0, S)
            pltpu.sync_copy(i_hbm.at[jax.lax.div(j0, ck)], idx_v)
            pltpu.sync_copy(x_hbm.at[pl.ds(src, ck)], data_v)
            pltpu.async_copy(data_v, o_hbm.at[idx_v], sem).wait()

    return kern(x, idx2d)


def _sc_gather(y, idx2d, n):
    P, d = y.shape
    nrows, ck = idx2d.shape
    per_w = n // _SC_NW
    nch = per_w // ck
    mesh = plsc.VectorSubcoreMesh(core_axis_name="c", subcore_axis_name="s")

    @functools.partial(
        pl.kernel, mesh=mesh,
        out_type=jax.ShapeDtypeStruct((n, d), y.dtype),
        scratch_types=[
            pltpu.VMEM((ck,), jnp.int32),
            pltpu.VMEM((ck, d), y.dtype),
            pltpu.SemaphoreType.DMA,
        ],
    )
    def kern(y_hbm, i_hbm, o_hbm, idx_v, rows_v, sem):
        wid = jax.lax.axis_index("s") * _SC_NC + jax.lax.axis_index("c")

        @pl.loop(0, nch)
        def _(c):
            j0 = wid * per_w + c * ck
            pltpu.sync_copy(i_hbm.at[jax.lax.div(j0, ck)], idx_v)
            pltpu.async_copy(y_hbm.at[idx_v], rows_v, sem).wait()
            pltpu.sync_copy(rows_v, o_hbm.at[pl.ds(j0, ck)])

    return kern(y, idx2d)


def _run_moe(blk, xg, W1, W3, W2):
    P, d = xg.shape
    E, _, F = W1.shape
    NB = P // MB
    fb = min(FB, F)
    NF = F // fb

    def x_map(b, f, blk_ref):
        valid = blk_ref[b] < E
        return (jnp.where(valid, b, 0), 0)

    def w13_map(b, f, blk_ref):
        e = blk_ref[b]
        valid = e < E
        return (jnp.minimum(e, E - 1), 0, jnp.where(valid, f, 0))

    def w2_map(b, f, blk_ref):
        e = blk_ref[b]
        valid = e < E
        return (jnp.minimum(e, E - 1), jnp.where(valid, f, 0), 0)

    grid_spec = pltpu.PrefetchScalarGridSpec(
        num_scalar_prefetch=1,
        grid=(NB, NF),
        in_specs=[
            pl.BlockSpec((MB, d), x_map),
            pl.BlockSpec((1, d, fb), w13_map),
            pl.BlockSpec((1, d, fb), w13_map),
            pl.BlockSpec((1, fb, d), w2_map),
        ],
        out_specs=pl.BlockSpec((MB, d), lambda b, f, blk_ref: (b, 0)),
        scratch_shapes=[pltpu.VMEM((MB, d), jnp.float32)],
    )
    return pl.pallas_call(
        functools.partial(_moe_kernel_body, E),
        grid_spec=grid_spec,
        out_shape=jax.ShapeDtypeStruct((P, d), jnp.float32),
        compiler_params=pltpu.CompilerParams(
            dimension_semantics=("arbitrary", "arbitrary"),
        ),
    )(blk, xg, W1, W3, W2)


def _moe_kernel_body(E, blk_ref, xg_ref, w1_ref, w3_ref, w2_ref, y_ref,
                     acc_ref):
    f = pl.program_id(1)
    nf = pl.num_programs(1)
    b = pl.program_id(0)
    e = blk_ref[b]

    @pl.when(f == 0)
    def _():
        acc_ref[...] = jnp.zeros_like(acc_ref)

    @pl.when(e < E)
    def _():
        xb = xg_ref[...].astype(jnp.bfloat16)
        a = jnp.dot(xb, w1_ref[0].astype(jnp.bfloat16),
                    preferred_element_type=jnp.float32)
        g = jnp.dot(xb, w3_ref[0].astype(jnp.bfloat16),
                    preferred_element_type=jnp.float32)
        h = (jnp.maximum(a, jnp.float32(0.0)) * g).astype(jnp.bfloat16)
        acc_ref[...] += jnp.dot(h, w2_ref[0].astype(jnp.bfloat16),
                                preferred_element_type=jnp.float32)

    @pl.when(f == nf - 1)
    def _():
        y_ref[...] = acc_ref[...]


def _combine_body(y0_ref, y1_ref, w0_ref, w1_ref, out_ref):
    out_ref[...] = y0_ref[...] * w0_ref[...] + y1_ref[...] * w1_ref[...]


def _run_combine(yg, w0, w1):
    n, d = yg.shape
    S = n // 2
    TB = 256
    return pl.pallas_call(
        _combine_body,
        grid=(S // TB,),
        in_specs=[
            pl.BlockSpec((TB, d), lambda i: (i, 0)),
            pl.BlockSpec((TB, d), lambda i: (i + S // TB, 0)),
            pl.BlockSpec((TB, 1), lambda i: (i, 0)),
            pl.BlockSpec((TB, 1), lambda i: (i, 0)),
        ],
        out_specs=pl.BlockSpec((TB, d), lambda i: (i, 0)),
        out_shape=jax.ShapeDtypeStruct((S, d), jnp.float32),
    )(yg, yg, w0, w1)


def kernel(hidden_states, Wg, W1, W3, W2):
    b, s, d = hidden_states.shape
    E = Wg.shape[1]
    S = b * s
    P = ((2 * S + (E - 1) * MB) // MB) * MB
    x = hidden_states.reshape(S, d)

    w0, w1, dest32, blk = _run_router(x, Wg)
    idx2d = dest32.reshape(-1, _SC_CHUNK)
    blk_flat = blk.reshape(-1)

    xg = _sc_scatter(x, idx2d, P)
    y = _run_moe(blk_flat, xg, W1, W3, W2)
    yg = _sc_gather(y, idx2d, 2 * S)
    out = _run_combine(yg, w0, w1)
    return out.reshape(b, s, d)


# speedup vs baseline: 3.0961x; 1.4434x over previous
import functools

import jax
import jax.numpy as jnp
from jax.experimental import pallas as pl
from jax.experimental.pallas import tpu as pltpu
from jax.experimental.pallas import tpu_sc as plsc

TOPK = 2
MB = 576
FB = 512
SC_WIN = 128


def _iota(shape, dim):
    return jax.lax.broadcasted_iota(jnp.int32, shape, dim)


def _col_to_qc(xcol, Q, C, precision=None):
    N = xcol.shape[0]
    M = (_iota((N, C), 0) % C == _iota((N, C), 1)).astype(jnp.float32)
    XM = xcol * M
    L = (_iota((Q, N), 1) // C == _iota((Q, N), 0)).astype(jnp.float32)
    return jax.lax.dot_general(L, XM, (((1,), (0,)), ((), ())),
                               precision=precision,
                               preferred_element_type=jnp.float32)


def _router_body(x_ref, wg_ref, w0_ref, w1_ref, dest_ref, blk_ref):
    S, E = x_ref.shape[0], wg_ref.shape[1]
    P = 2 * S
    Q = S // 128
    NB = E + max(0, 2 * S - E) // MB
    HI = jax.lax.Precision.HIGHEST

    x = x_ref[...]

    logits = jax.lax.dot_general(x, wg_ref[...], (((1,), (0,)), ((), ())),
                                 preferred_element_type=jnp.float32)
    m = jnp.max(logits, axis=1, keepdims=True)
    p = jnp.exp(logits - m)
    p = p / jnp.sum(p, axis=1, keepdims=True)
    lane = _iota((S, E), 1)
    i0 = jnp.min(jnp.where(logits == m, lane, E), axis=1, keepdims=True)
    l2 = jnp.where(lane == i0, -jnp.inf, logits)
    m1 = jnp.max(l2, axis=1, keepdims=True)
    i1 = jnp.min(jnp.where(l2 == m1, lane, E), axis=1, keepdims=True)
    oh0 = (lane == i0)
    oh1 = (lane == i1)
    w0_ref[...] = jnp.sum(jnp.where(oh0, p, 0.0), axis=1, keepdims=True)
    w1_ref[...] = jnp.sum(jnp.where(oh1, p, 0.0), axis=1, keepdims=True)

    i0qc = _col_to_qc(i0.astype(jnp.float32), Q, 128)
    i1qc = _col_to_qc(i1.astype(jnp.float32), Q, 128)
    ep = jnp.concatenate([i0qc, i1qc], axis=0)

    R = E * 2 * Q
    T = (_iota((R, 2 * Q), 0) % (2 * Q) == _iota((R, 2 * Q), 1))
    ep_t = jax.lax.dot_general(T.astype(jnp.float32), ep,
                               (((1,), (0,)), ((), ())),
                               preferred_element_type=jnp.float32)
    e_of_r = (_iota((R, 128), 0) // (2 * Q)).astype(jnp.float32)
    A2 = (ep_t == e_of_r).astype(jnp.float32)

    U128 = (_iota((128, 128), 0) <= _iota((128, 128), 1)).astype(jnp.float32)
    inrow = jax.lax.dot_general(A2, U128, (((1,), (0,)), ((), ())),
                                preferred_element_type=jnp.float32)
    rowtot = inrow[:, 127:128]

    rr = _iota((R, R), 0)
    cc = _iota((R, R), 1)
    same_e = (rr // (2 * Q)) == (cc // (2 * Q))
    prev_row = same_e & ((cc % (2 * Q)) < (rr % (2 * Q)))
    Pexcl = jax.lax.dot_general(prev_row.astype(jnp.float32), rowtot,
                                (((1,), (0,)), ((), ())),
                                preferred_element_type=jnp.float32)
    counts_pg = jax.lax.dot_general(same_e.astype(jnp.float32), rowtot,
                                    (((1,), (0,)), ((), ())),
                                    preferred_element_type=jnp.float32)
    pc_pg = jnp.floor((counts_pg + (MB - 1)) / MB) * MB
    prev_e = ((cc // (2 * Q)) < (rr // (2 * Q))).astype(jnp.float32)
    base_pg = jax.lax.dot_general(prev_e, pc_pg, (((1,), (0,)), ((), ())),
                                  preferred_element_type=jnp.float32)
    base_pg = base_pg / (2 * Q)

    cum = inrow + Pexcl
    dest_pg = (base_pg + cum - 1.0) * A2

    G = (_iota((2 * Q, R), 1) % (2 * Q) == _iota((2 * Q, R), 0))
    dest32 = jax.lax.dot_general(G.astype(jnp.float32), dest_pg,
                                 (((1,), (0,)), ((), ())),
                                 precision=HI, preferred_element_type=jnp.float32)
    dest_ref[...] = dest32.astype(jnp.int32)

    sel = (_iota((R, E), 0) == _iota((R, E), 1) * (2 * Q)).astype(jnp.float32)
    base8 = jax.lax.dot_general(sel, base_pg, (((0,), (0,)), ((), ())),
                                preferred_element_type=jnp.float32)
    pc8 = jax.lax.dot_general(sel, pc_pg, (((0,), (0,)), ((), ())),
                              preferred_element_type=jnp.float32)
    border = (_iota((E, NB), 1) * MB).astype(jnp.float32)
    owns = ((base8 <= border) & (border < base8 + pc8)).astype(jnp.float32)
    e_col = _iota((E, 1), 0).astype(jnp.float32)
    blk_e = jnp.sum(owns * e_col, axis=0, keepdims=True)
    blk_v = jnp.sum(owns, axis=0, keepdims=True)
    blk_ref[...] = jnp.where(blk_v > 0.5, blk_e, jnp.float32(E)).astype(jnp.int32)


def _run_router(x, Wg):
    S, d = x.shape
    E = Wg.shape[1]
    NB = E + max(0, 2 * S - E) // MB
    return pl.pallas_call(
        _router_body,
        out_shape=(
            jax.ShapeDtypeStruct((S, 1), jnp.float32),
            jax.ShapeDtypeStruct((S, 1), jnp.float32),
            jax.ShapeDtypeStruct((2 * (S // 128), 128), jnp.int32),
            jax.ShapeDtypeStruct((1, NB), jnp.int32),
        ),
    )(x, Wg)


_SC_NC = 2
_SC_NS = 16
_SC_NW = _SC_NC * _SC_NS
_SC_CHUNK = 32


def _sc_scatter(x, idx2d, P):
    S, d = x.shape
    nrows, ck = idx2d.shape
    n = nrows * ck
    per_w = n // _SC_NW
    nch = per_w // ck
    mesh = plsc.VectorSubcoreMesh(core_axis_name="c", subcore_axis_name="s")

    @functools.partial(
        pl.kernel, mesh=mesh,
        out_type=jax.ShapeDtypeStruct((P, d), x.dtype),
        scratch_types=[
            pltpu.VMEM((ck,), jnp.int32),
            pltpu.VMEM((ck, d), x.dtype),
            pltpu.SemaphoreType.DMA,
        ],
    )
    def kern(x_hbm, i_hbm, o_hbm, idx_v, data_v, sem):
        wid = jax.lax.axis_index("s") * _SC_NC + jax.lax.axis_index("c")

        @pl.loop(0, nch)
        def _(c):
            j0 = wid * per_w + c * ck
            src = jax.lax.rem(j0, S)
            pltpu.sync_copy(i_hbm.at[jax.lax.div(j0, ck)], idx_v)
            pltpu.sync_copy(x_hbm.at[pl.ds(src, ck)], data_v)
            pltpu.async_copy(data_v, o_hbm.at[idx_v], sem).wait()

    return kern(x, idx2d)


def _sc_gather(y, idx2d, n):
    P, d = y.shape
    nrows, ck = idx2d.shape
    per_w = n // _SC_NW
    nch = per_w // ck
    mesh = plsc.VectorSubcoreMesh(core_axis_name="c", subcore_axis_name="s")

    @functools.partial(
        pl.kernel, mesh=mesh,
        out_type=jax.ShapeDtypeStruct((n, d), y.dtype),
        scratch_types=[
            pltpu.VMEM((ck,), jnp.int32),
            pltpu.VMEM((ck, d), y.dtype),
            pltpu.SemaphoreType.DMA,
        ],
    )
    def kern(y_hbm, i_hbm, o_hbm, idx_v, rows_v, sem):
        wid = jax.lax.axis_index("s") * _SC_NC + jax.lax.axis_index("c")

        @pl.loop(0, nch)
        def _(c):
            j0 = wid * per_w + c * ck
            pltpu.sync_copy(i_hbm.at[jax.lax.div(j0, ck)], idx_v)
            pltpu.async_copy(y_hbm.at[idx_v], rows_v, sem).wait()
            pltpu.sync_copy(rows_v, o_hbm.at[pl.ds(j0, ck)])

    return kern(y, idx2d)


def _run_moe(blk, xg, W1, W3, W2):
    P, d = xg.shape
    E, _, F = W1.shape
    NB = P // MB
    fb = min(FB, F)
    NF = F // fb

    def x_map(b, f, blk_ref):
        valid = blk_ref[b] < E
        return (jnp.where(valid, b, 0), 0)

    def w13_map(b, f, blk_ref):
        e = blk_ref[b]
        valid = e < E
        return (jnp.minimum(e, E - 1), 0, jnp.where(valid, f, 0))

    def w2_map(b, f, blk_ref):
        e = blk_ref[b]
        valid = e < E
        return (jnp.minimum(e, E - 1), jnp.where(valid, f, 0), 0)

    grid_spec = pltpu.PrefetchScalarGridSpec(
        num_scalar_prefetch=1,
        grid=(NB, NF),
        in_specs=[
            pl.BlockSpec((MB, d), x_map),
            pl.BlockSpec((1, d, fb), w13_map),
            pl.BlockSpec((1, d, fb), w13_map),
            pl.BlockSpec((1, fb, d), w2_map),
        ],
        out_specs=pl.BlockSpec((MB, d), lambda b, f, blk_ref: (b, 0)),
        scratch_shapes=[pltpu.VMEM((MB, d), jnp.float32)],
    )
    return pl.pallas_call(
        functools.partial(_moe_kernel_body, E),
        grid_spec=grid_spec,
        out_shape=jax.ShapeDtypeStruct((P, d), jnp.float32),
        compiler_params=pltpu.CompilerParams(
            dimension_semantics=("arbitrary", "arbitrary"),
        ),
    )(blk, xg, W1, W3, W2)


def _moe_kernel_body(E, blk_ref, xg_ref, w1_ref, w3_ref, w2_ref, y_ref,
                     acc_ref):
    f = pl.program_id(1)
    nf = pl.num_programs(1)
    b = pl.program_id(0)
    e = blk_ref[b]

    @pl.when(f == 0)
    def _():
        acc_ref[...] = jnp.zeros_like(acc_ref)

    @pl.when(e < E)
    def _():
        xb = xg_ref[...].astype(jnp.bfloat16)
        a = jnp.dot(xb, w1_ref[0].astype(jnp.bfloat16),
                    preferred_element_type=jnp.float32)
        g = jnp.dot(xb, w3_ref[0].astype(jnp.bfloat16),
                    preferred_element_type=jnp.float32)
        h = (jnp.maximum(a, jnp.float32(0.0)) * g).astype(jnp.bfloat16)
        acc_ref[...] += jnp.dot(h, w2_ref[0].astype(jnp.bfloat16),
                                preferred_element_type=jnp.float32)

    @pl.when(f == nf - 1)
    def _():
        y_ref[...] = acc_ref[...]


def _combine_body(y0_ref, y1_ref, w0_ref, w1_ref, out_ref):
    out_ref[...] = y0_ref[...] * w0_ref[...] + y1_ref[...] * w1_ref[...]


def _run_combine(yg, w0, w1):
    n, d = yg.shape
    S = n // 2
    TB = 256
    return pl.pallas_call(
        _combine_body,
        grid=(S // TB,),
        in_specs=[
            pl.BlockSpec((TB, d), lambda i: (i, 0)),
            pl.BlockSpec((TB, d), lambda i: (i + S // TB, 0)),
            pl.BlockSpec((TB, 1), lambda i: (i, 0)),
            pl.BlockSpec((TB, 1), lambda i: (i, 0)),
        ],
        out_specs=pl.BlockSpec((TB, d), lambda i: (i, 0)),
        out_shape=jax.ShapeDtypeStruct((S, d), jnp.float32),
    )(yg, yg, w0, w1)


def kernel(hidden_states, Wg, W1, W3, W2):
    b, s, d = hidden_states.shape
    E = Wg.shape[1]
    S = b * s
    P = (E + max(0, 2 * S - E) // MB) * MB
    x = hidden_states.reshape(S, d)

    w0, w1, dest32, blk = _run_router(x, Wg)
    idx2d = dest32.reshape(-1, _SC_CHUNK)
    blk_flat = blk.reshape(-1)

    xg = _sc_scatter(x, idx2d, P)
    y = _run_moe(blk_flat, xg, W1, W3, W2)
    yg = _sc_gather(y, idx2d, 2 * S)
    out = _run_combine(yg, w0, w1)
    return out.reshape(b, s, d)


# speedup vs baseline: 3.1055x; 1.0030x over previous
import functools

import jax
import jax.numpy as jnp
from jax.experimental import pallas as pl
from jax.experimental.pallas import tpu as pltpu
from jax.experimental.pallas import tpu_sc as plsc

TOPK = 2
MB = 576
FB = 512
SC_WIN = 128


def _iota(shape, dim):
    return jax.lax.broadcasted_iota(jnp.int32, shape, dim)


def _col_to_qc(xcol, Q, C, precision=None):
    N = xcol.shape[0]
    M = (_iota((N, C), 0) % C == _iota((N, C), 1)).astype(jnp.float32)
    XM = xcol * M
    L = (_iota((Q, N), 1) // C == _iota((Q, N), 0)).astype(jnp.float32)
    return jax.lax.dot_general(L, XM, (((1,), (0,)), ((), ())),
                               precision=precision,
                               preferred_element_type=jnp.float32)


def _router_body(x_ref, wg_ref, w0_ref, w1_ref, dest_ref, blk_ref):
    S, E = x_ref.shape[0], wg_ref.shape[1]
    P = 2 * S
    Q = S // 128
    NB = E + max(0, 2 * S - E) // MB
    HI = jax.lax.Precision.HIGHEST

    x = x_ref[...]

    logits = jax.lax.dot_general(x, wg_ref[...], (((1,), (0,)), ((), ())),
                                 preferred_element_type=jnp.float32)
    m = jnp.max(logits, axis=1, keepdims=True)
    p = jnp.exp(logits - m)
    p = p / jnp.sum(p, axis=1, keepdims=True)
    lane = _iota((S, E), 1)
    i0 = jnp.min(jnp.where(logits == m, lane, E), axis=1, keepdims=True)
    l2 = jnp.where(lane == i0, -jnp.inf, logits)
    m1 = jnp.max(l2, axis=1, keepdims=True)
    i1 = jnp.min(jnp.where(l2 == m1, lane, E), axis=1, keepdims=True)
    oh0 = (lane == i0)
    oh1 = (lane == i1)
    w0_ref[...] = jnp.sum(jnp.where(oh0, p, 0.0), axis=1, keepdims=True)
    w1_ref[...] = jnp.sum(jnp.where(oh1, p, 0.0), axis=1, keepdims=True)

    i0qc = _col_to_qc(i0.astype(jnp.float32), Q, 128)
    i1qc = _col_to_qc(i1.astype(jnp.float32), Q, 128)
    ep = jnp.concatenate([i0qc, i1qc], axis=0)

    R = E * 2 * Q
    T = (_iota((R, 2 * Q), 0) % (2 * Q) == _iota((R, 2 * Q), 1))
    ep_t = jax.lax.dot_general(T.astype(jnp.float32), ep,
                               (((1,), (0,)), ((), ())),
                               preferred_element_type=jnp.float32)
    e_of_r = (_iota((R, 128), 0) // (2 * Q)).astype(jnp.float32)
    A2 = (ep_t == e_of_r).astype(jnp.float32)

    U128 = (_iota((128, 128), 0) <= _iota((128, 128), 1)).astype(jnp.float32)
    inrow = jax.lax.dot_general(A2, U128, (((1,), (0,)), ((), ())),
                                preferred_element_type=jnp.float32)
    rowtot = inrow[:, 127:128]

    rr = _iota((R, R), 0)
    cc = _iota((R, R), 1)
    same_e = (rr // (2 * Q)) == (cc // (2 * Q))
    prev_row = same_e & ((cc % (2 * Q)) < (rr % (2 * Q)))
    Pexcl = jax.lax.dot_general(prev_row.astype(jnp.float32), rowtot,
                                (((1,), (0,)), ((), ())),
                                preferred_element_type=jnp.float32)
    counts_pg = jax.lax.dot_general(same_e.astype(jnp.float32), rowtot,
                                    (((1,), (0,)), ((), ())),
                                    preferred_element_type=jnp.float32)
    pc_pg = jnp.floor((counts_pg + (MB - 1)) / MB) * MB
    prev_e = ((cc // (2 * Q)) < (rr // (2 * Q))).astype(jnp.float32)
    base_pg = jax.lax.dot_general(prev_e, pc_pg, (((1,), (0,)), ((), ())),
                                  preferred_element_type=jnp.float32)
    base_pg = base_pg / (2 * Q)

    cum = inrow + Pexcl
    dest_pg = (base_pg + cum - 1.0) * A2

    G = (_iota((2 * Q, R), 1) % (2 * Q) == _iota((2 * Q, R), 0))
    dest32 = jax.lax.dot_general(G.astype(jnp.float32), dest_pg,
                                 (((1,), (0,)), ((), ())),
                                 precision=HI, preferred_element_type=jnp.float32)
    dest_ref[...] = dest32.astype(jnp.int32)

    sel = (_iota((R, E), 0) == _iota((R, E), 1) * (2 * Q)).astype(jnp.float32)
    base8 = jax.lax.dot_general(sel, base_pg, (((0,), (0,)), ((), ())),
                                preferred_element_type=jnp.float32)
    pc8 = jax.lax.dot_general(sel, pc_pg, (((0,), (0,)), ((), ())),
                              preferred_element_type=jnp.float32)
    border = (_iota((E, NB), 1) * MB).astype(jnp.float32)
    owns = ((base8 <= border) & (border < base8 + pc8)).astype(jnp.float32)
    e_col = _iota((E, 1), 0).astype(jnp.float32)
    blk_e = jnp.sum(owns * e_col, axis=0, keepdims=True)
    blk_v = jnp.sum(owns, axis=0, keepdims=True)
    blk_ref[...] = jnp.where(blk_v > 0.5, blk_e, jnp.float32(E)).astype(jnp.int32)


def _run_router(x, Wg):
    S, d = x.shape
    E = Wg.shape[1]
    NB = E + max(0, 2 * S - E) // MB
    return pl.pallas_call(
        _router_body,
        out_shape=(
            jax.ShapeDtypeStruct((S, 1), jnp.float32),
            jax.ShapeDtypeStruct((S, 1), jnp.float32),
            jax.ShapeDtypeStruct((2 * (S // 128), 128), jnp.int32),
            jax.ShapeDtypeStruct((1, NB), jnp.int32),
        ),
    )(x, Wg)


_SC_NC = 2
_SC_NS = 16
_SC_NW = _SC_NC * _SC_NS
_SC_CHUNK = 32


def _sc_scatter(x, idx2d, P):
    S, d = x.shape
    nrows, ck = idx2d.shape
    n = nrows * ck
    per_w = n // _SC_NW
    nch = per_w // ck
    mesh = plsc.VectorSubcoreMesh(core_axis_name="c", subcore_axis_name="s")

    @functools.partial(
        pl.kernel, mesh=mesh,
        out_type=jax.ShapeDtypeStruct((P, d), x.dtype),
        scratch_types=[
            pltpu.VMEM((ck,), jnp.int32),
            pltpu.VMEM((ck, d), x.dtype),
            pltpu.SemaphoreType.DMA,
        ],
    )
    def kern(x_hbm, i_hbm, o_hbm, idx_v, data_v, sem):
        wid = jax.lax.axis_index("s") * _SC_NC + jax.lax.axis_index("c")

        @pl.loop(0, nch)
        def _(c):
            j0 = wid * per_w + c * ck
            src = jax.lax.rem(j0, S)
            pltpu.sync_copy(i_hbm.at[jax.lax.div(j0, ck)], idx_v)
            pltpu.sync_copy(x_hbm.at[pl.ds(src, ck)], data_v)
            pltpu.async_copy(data_v, o_hbm.at[idx_v], sem).wait()

    return kern(x, idx2d)


def _sc_gather(y, idx2d, n):
    P, d = y.shape
    nrows, ck = idx2d.shape
    per_w = n // _SC_NW
    nch = per_w // ck
    mesh = plsc.VectorSubcoreMesh(core_axis_name="c", subcore_axis_name="s")

    @functools.partial(
        pl.kernel, mesh=mesh,
        out_type=jax.ShapeDtypeStruct((n, d), y.dtype),
        scratch_types=[
            pltpu.VMEM((ck,), jnp.int32),
            pltpu.VMEM((ck, d), y.dtype),
            pltpu.SemaphoreType.DMA,
        ],
    )
    def kern(y_hbm, i_hbm, o_hbm, idx_v, rows_v, sem):
        wid = jax.lax.axis_index("s") * _SC_NC + jax.lax.axis_index("c")

        @pl.loop(0, nch)
        def _(c):
            j0 = wid * per_w + c * ck
            pltpu.sync_copy(i_hbm.at[jax.lax.div(j0, ck)], idx_v)
            pltpu.async_copy(y_hbm.at[idx_v], rows_v, sem).wait()
            pltpu.sync_copy(rows_v, o_hbm.at[pl.ds(j0, ck)])

    return kern(y, idx2d)


def _run_moe(blk, xg, W1, W3, W2):
    P, d = xg.shape
    E, _, F = W1.shape
    NB = P // MB
    fb = min(FB, F)
    NF = F // fb

    def x_map(b, f, blk_ref):
        valid = blk_ref[b] < E
        return (jnp.where(valid, b, 0), 0)

    def w13_map(b, f, blk_ref):
        e = blk_ref[b]
        valid = e < E
        return (jnp.minimum(e, E - 1), 0, jnp.where(valid, f, 0))

    def w2_map(b, f, blk_ref):
        e = blk_ref[b]
        valid = e < E
        return (jnp.minimum(e, E - 1), jnp.where(valid, f, 0), 0)

    grid_spec = pltpu.PrefetchScalarGridSpec(
        num_scalar_prefetch=1,
        grid=(NB, NF),
        in_specs=[
            pl.BlockSpec((MB, d), x_map),
            pl.BlockSpec((1, d, fb), w13_map),
            pl.BlockSpec((1, d, fb), w13_map),
            pl.BlockSpec((1, fb, d), w2_map),
        ],
        out_specs=pl.BlockSpec((MB, d), lambda b, f, blk_ref: (b, 0)),
        scratch_shapes=[pltpu.VMEM((MB, d), jnp.float32)],
    )
    return pl.pallas_call(
        functools.partial(_moe_kernel_body, E),
        grid_spec=grid_spec,
        out_shape=jax.ShapeDtypeStruct((P, d), jnp.float32),
        compiler_params=pltpu.CompilerParams(
            dimension_semantics=("arbitrary", "arbitrary"),
        ),
    )(blk, xg, W1, W3, W2)


def _moe_kernel_body(E, blk_ref, xg_ref, w1_ref, w3_ref, w2_ref, y_ref,
                     acc_ref):
    f = pl.program_id(1)
    nf = pl.num_programs(1)
    b = pl.program_id(0)
    e = blk_ref[b]

    @pl.when(f == 0)
    def _():
        acc_ref[...] = jnp.zeros_like(acc_ref)

    @pl.when(e < E)
    def _():
        xb = xg_ref[...]
        a = jnp.dot(xb, w1_ref[0], preferred_element_type=jnp.float32)
        g = jnp.dot(xb, w3_ref[0], preferred_element_type=jnp.float32)
        h = jnp.maximum(a, jnp.float32(0.0)) * g
        acc_ref[...] += jnp.dot(h, w2_ref[0],
                                preferred_element_type=jnp.float32)

    @pl.when(f == nf - 1)
    def _():
        y_ref[...] = acc_ref[...]


def _combine_body(y0_ref, y1_ref, w0_ref, w1_ref, out_ref):
    out_ref[...] = y0_ref[...] * w0_ref[...] + y1_ref[...] * w1_ref[...]


def _run_combine(yg, w0, w1):
    n, d = yg.shape
    S = n // 2
    TB = 256
    return pl.pallas_call(
        _combine_body,
        grid=(S // TB,),
        in_specs=[
            pl.BlockSpec((TB, d), lambda i: (i, 0)),
            pl.BlockSpec((TB, d), lambda i: (i + S // TB, 0)),
            pl.BlockSpec((TB, 1), lambda i: (i, 0)),
            pl.BlockSpec((TB, 1), lambda i: (i, 0)),
        ],
        out_specs=pl.BlockSpec((TB, d), lambda i: (i, 0)),
        out_shape=jax.ShapeDtypeStruct((S, d), jnp.float32),
    )(yg, yg, w0, w1)


def kernel(hidden_states, Wg, W1, W3, W2):
    b, s, d = hidden_states.shape
    E = Wg.shape[1]
    S = b * s
    P = (E + max(0, 2 * S - E) // MB) * MB
    x = hidden_states.reshape(S, d)

    w0, w1, dest32, blk = _run_router(x, Wg)
    idx2d = dest32.reshape(-1, _SC_CHUNK)
    blk_flat = blk.reshape(-1)

    xg = _sc_scatter(x, idx2d, P)
    y = _run_moe(blk_flat, xg, W1, W3, W2)
    yg = _sc_gather(y, idx2d, 2 * S)
    out = _run_combine(yg, w0, w1)
    return out.reshape(b, s, d)


# speedup vs baseline: 3.1929x; 1.0282x over previous
import functools

import jax
import jax.numpy as jnp
from jax.experimental import pallas as pl
from jax.experimental.pallas import tpu as pltpu
from jax.experimental.pallas import tpu_sc as plsc

TOPK = 2
MB = 576
FB = 512
SC_WIN = 128


def _iota(shape, dim):
    return jax.lax.broadcasted_iota(jnp.int32, shape, dim)


def _col_to_qc(xcol, Q, C, precision=None):
    N = xcol.shape[0]
    M = (_iota((N, C), 0) % C == _iota((N, C), 1)).astype(jnp.float32)
    XM = xcol * M
    L = (_iota((Q, N), 1) // C == _iota((Q, N), 0)).astype(jnp.float32)
    return jax.lax.dot_general(L, XM, (((1,), (0,)), ((), ())),
                               precision=precision,
                               preferred_element_type=jnp.float32)


def _router_body(x_ref, wg_ref, w0_ref, w1_ref, dest_ref, blk_ref):
    S, E = x_ref.shape[0], wg_ref.shape[1]
    P = 2 * S
    Q = S // 128
    NB = E + max(0, 2 * S - E) // MB
    HI = jax.lax.Precision.HIGHEST

    x = x_ref[...]

    logits = jax.lax.dot_general(x, wg_ref[...], (((1,), (0,)), ((), ())),
                                 preferred_element_type=jnp.float32)
    m = jnp.max(logits, axis=1, keepdims=True)
    p = jnp.exp(logits - m)
    p = p / jnp.sum(p, axis=1, keepdims=True)
    lane = _iota((S, E), 1)
    i0 = jnp.min(jnp.where(logits == m, lane, E), axis=1, keepdims=True)
    l2 = jnp.where(lane == i0, -jnp.inf, logits)
    m1 = jnp.max(l2, axis=1, keepdims=True)
    i1 = jnp.min(jnp.where(l2 == m1, lane, E), axis=1, keepdims=True)
    oh0 = (lane == i0)
    oh1 = (lane == i1)
    w0_ref[...] = jnp.sum(jnp.where(oh0, p, 0.0), axis=1, keepdims=True)
    w1_ref[...] = jnp.sum(jnp.where(oh1, p, 0.0), axis=1, keepdims=True)

    i0qc = _col_to_qc(i0.astype(jnp.float32), Q, 128)
    i1qc = _col_to_qc(i1.astype(jnp.float32), Q, 128)
    ep = jnp.concatenate([i0qc, i1qc], axis=0)

    R = E * 2 * Q
    T = (_iota((R, 2 * Q), 0) % (2 * Q) == _iota((R, 2 * Q), 1))
    ep_t = jax.lax.dot_general(T.astype(jnp.float32), ep,
                               (((1,), (0,)), ((), ())),
                               preferred_element_type=jnp.float32)
    e_of_r = (_iota((R, 128), 0) // (2 * Q)).astype(jnp.float32)
    A2 = (ep_t == e_of_r).astype(jnp.float32)

    U128 = (_iota((128, 128), 0) <= _iota((128, 128), 1)).astype(jnp.float32)
    inrow = jax.lax.dot_general(A2, U128, (((1,), (0,)), ((), ())),
                                preferred_element_type=jnp.float32)
    rowtot = inrow[:, 127:128]

    rr = _iota((R, R), 0)
    cc = _iota((R, R), 1)
    same_e = (rr // (2 * Q)) == (cc // (2 * Q))
    prev_row = same_e & ((cc % (2 * Q)) < (rr % (2 * Q)))
    Pexcl = jax.lax.dot_general(prev_row.astype(jnp.float32), rowtot,
                                (((1,), (0,)), ((), ())),
                                preferred_element_type=jnp.float32)
    counts_pg = jax.lax.dot_general(same_e.astype(jnp.float32), rowtot,
                                    (((1,), (0,)), ((), ())),
                                    preferred_element_type=jnp.float32)
    pc_pg = jnp.floor((counts_pg + (MB - 1)) / MB) * MB
    prev_e = ((cc // (2 * Q)) < (rr // (2 * Q))).astype(jnp.float32)
    base_pg = jax.lax.dot_general(prev_e, pc_pg, (((1,), (0,)), ((), ())),
                                  preferred_element_type=jnp.float32)
    base_pg = base_pg / (2 * Q)

    cum = inrow + Pexcl
    dest_pg = (base_pg + cum - 1.0) * A2

    G = (_iota((2 * Q, R), 1) % (2 * Q) == _iota((2 * Q, R), 0))
    dest32 = jax.lax.dot_general(G.astype(jnp.float32), dest_pg,
                                 (((1,), (0,)), ((), ())),
                                 precision=HI, preferred_element_type=jnp.float32)
    dest_ref[...] = dest32.astype(jnp.int32)

    sel = (_iota((R, E), 0) == _iota((R, E), 1) * (2 * Q)).astype(jnp.float32)
    base8 = jax.lax.dot_general(sel, base_pg, (((0,), (0,)), ((), ())),
                                preferred_element_type=jnp.float32)
    pc8 = jax.lax.dot_general(sel, pc_pg, (((0,), (0,)), ((), ())),
                              preferred_element_type=jnp.float32)
    border = (_iota((E, NB), 1) * MB).astype(jnp.float32)
    owns = ((base8 <= border) & (border < base8 + pc8)).astype(jnp.float32)
    e_col = _iota((E, 1), 0).astype(jnp.float32)
    blk_e = jnp.sum(owns * e_col, axis=0, keepdims=True)
    blk_v = jnp.sum(owns, axis=0, keepdims=True)
    blk_ref[...] = jnp.where(blk_v > 0.5, blk_e, jnp.float32(E)).astype(jnp.int32)


def _run_router(x, Wg):
    S, d = x.shape
    E = Wg.shape[1]
    NB = E + max(0, 2 * S - E) // MB
    return pl.pallas_call(
        _router_body,
        out_shape=(
            jax.ShapeDtypeStruct((S, 1), jnp.float32),
            jax.ShapeDtypeStruct((S, 1), jnp.float32),
            jax.ShapeDtypeStruct((2 * (S // 128), 128), jnp.int32),
            jax.ShapeDtypeStruct((1, NB), jnp.int32),
        ),
    )(x, Wg)


_SC_NC = 2
_SC_NS = 16
_SC_NW = _SC_NC * _SC_NS
_SC_CHUNK = 32


def _sc_scatter(x, idx2d, P):
    S, d = x.shape
    nrows, ck = idx2d.shape
    n = nrows * ck
    per_w = n // _SC_NW
    nch = per_w // ck
    mesh = plsc.VectorSubcoreMesh(core_axis_name="c", subcore_axis_name="s")

    @functools.partial(
        pl.kernel, mesh=mesh,
        out_type=jax.ShapeDtypeStruct((P, d), x.dtype),
        scratch_types=[
            pltpu.VMEM((ck,), jnp.int32),
            pltpu.VMEM((ck, d), x.dtype),
            pltpu.SemaphoreType.DMA,
        ],
    )
    def kern(x_hbm, i_hbm, o_hbm, idx_v, data_v, sem):
        wid = jax.lax.axis_index("s") * _SC_NC + jax.lax.axis_index("c")

        @pl.loop(0, nch)
        def _(c):
            j0 = wid * per_w + c * ck
            src = jax.lax.rem(j0, S)
            pltpu.sync_copy(i_hbm.at[jax.lax.div(j0, ck)], idx_v)
            pltpu.sync_copy(x_hbm.at[pl.ds(src, ck)], data_v)
            pltpu.async_copy(data_v, o_hbm.at[idx_v], sem).wait()

    return kern(x, idx2d)


def _sc_gather(y, idx2d, n):
    P, d = y.shape
    nrows, ck = idx2d.shape
    per_w = n // _SC_NW
    nch = per_w // ck
    mesh = plsc.VectorSubcoreMesh(core_axis_name="c", subcore_axis_name="s")

    @functools.partial(
        pl.kernel, mesh=mesh,
        out_type=jax.ShapeDtypeStruct((n, d), y.dtype),
        scratch_types=[
            pltpu.VMEM((ck,), jnp.int32),
            pltpu.VMEM((ck, d), y.dtype),
            pltpu.SemaphoreType.DMA,
        ],
    )
    def kern(y_hbm, i_hbm, o_hbm, idx_v, rows_v, sem):
        wid = jax.lax.axis_index("s") * _SC_NC + jax.lax.axis_index("c")

        @pl.loop(0, nch)
        def _(c):
            j0 = wid * per_w + c * ck
            pltpu.sync_copy(i_hbm.at[jax.lax.div(j0, ck)], idx_v)
            pltpu.async_copy(y_hbm.at[idx_v], rows_v, sem).wait()
            pltpu.sync_copy(rows_v, o_hbm.at[pl.ds(j0, ck)])

    return kern(y, idx2d)


def _run_moe(blk, xg, W1, W3, W2):
    P, d = xg.shape
    E, _, F = W1.shape
    NB = P // MB
    fb = min(FB, F)
    NF = F // fb

    def x_map(b, f, blk_ref):
        valid = blk_ref[b] < E
        return (jnp.where(valid, b, 0), 0)

    def w13_map(b, f, blk_ref):
        e = blk_ref[b]
        valid = e < E
        return (jnp.minimum(e, E - 1), 0, jnp.where(valid, f, 0))

    def w2_map(b, f, blk_ref):
        e = blk_ref[b]
        valid = e < E
        return (jnp.minimum(e, E - 1), jnp.where(valid, f, 0), 0)

    grid_spec = pltpu.PrefetchScalarGridSpec(
        num_scalar_prefetch=1,
        grid=(NB, NF),
        in_specs=[
            pl.BlockSpec((MB, d), x_map),
            pl.BlockSpec((1, d, fb), w13_map),
            pl.BlockSpec((1, d, fb), w13_map),
            pl.BlockSpec((1, fb, d), w2_map),
        ],
        out_specs=pl.BlockSpec((MB, d // 2), lambda b, f, blk_ref: (b, 0)),
        scratch_shapes=[pltpu.VMEM((MB, d), jnp.float32)],
    )
    return pl.pallas_call(
        functools.partial(_moe_kernel_body, E),
        grid_spec=grid_spec,
        out_shape=jax.ShapeDtypeStruct((P, d // 2), jnp.int32),
        compiler_params=pltpu.CompilerParams(
            dimension_semantics=("arbitrary", "arbitrary"),
        ),
    )(blk, xg, W1, W3, W2)


def _moe_kernel_body(E, blk_ref, xg_ref, w1_ref, w3_ref, w2_ref, y_ref,
                     acc_ref):
    f = pl.program_id(1)
    nf = pl.num_programs(1)
    b = pl.program_id(0)
    e = blk_ref[b]

    @pl.when(f == 0)
    def _():
        acc_ref[...] = jnp.zeros_like(acc_ref)

    @pl.when(e < E)
    def _():
        xb = xg_ref[...]
        a = jnp.dot(xb, w1_ref[0], preferred_element_type=jnp.float32)
        g = jnp.dot(xb, w3_ref[0], preferred_element_type=jnp.float32)
        h = jnp.maximum(a, jnp.float32(0.0)) * g
        acc_ref[...] += jnp.dot(h, w2_ref[0],
                                preferred_element_type=jnp.float32)

    @pl.when(f == nf - 1)
    def _():
        acc = acc_ref[...]
        hd = acc.shape[1] // 2
        lo = jax.lax.bitcast_convert_type(
            acc[:, :hd].astype(jnp.bfloat16), jnp.int16).astype(jnp.int32)
        hi = jax.lax.bitcast_convert_type(
            acc[:, hd:].astype(jnp.bfloat16), jnp.int16).astype(jnp.int32)
        y_ref[...] = (lo & 0xFFFF) | (hi << 16)


def _unpack(v):
    lo = jax.lax.bitcast_convert_type(
        (v & 0xFFFF).astype(jnp.int16), jnp.bfloat16).astype(jnp.float32)
    hi = jax.lax.bitcast_convert_type(
        jax.lax.shift_right_logical(v, 16).astype(jnp.int16),
        jnp.bfloat16).astype(jnp.float32)
    return lo, hi


def _combine_body(y0_ref, y1_ref, w0_ref, w1_ref, out_ref):
    lo0, hi0 = _unpack(y0_ref[...])
    lo1, hi1 = _unpack(y1_ref[...])
    hd = lo0.shape[1]
    w0 = w0_ref[...]
    w1 = w1_ref[...]
    out_ref[:, :hd] = lo0 * w0 + lo1 * w1
    out_ref[:, hd:] = hi0 * w0 + hi1 * w1


def _run_combine(yg, w0, w1):
    n, hd = yg.shape
    d = 2 * hd
    S = n // 2
    TB = 256
    return pl.pallas_call(
        _combine_body,
        grid=(S // TB,),
        in_specs=[
            pl.BlockSpec((TB, hd), lambda i: (i, 0)),
            pl.BlockSpec((TB, hd), lambda i: (i + S // TB, 0)),
            pl.BlockSpec((TB, 1), lambda i: (i, 0)),
            pl.BlockSpec((TB, 1), lambda i: (i, 0)),
        ],
        out_specs=pl.BlockSpec((TB, d), lambda i: (i, 0)),
        out_shape=jax.ShapeDtypeStruct((S, d), jnp.float32),
    )(yg, yg, w0, w1)


def kernel(hidden_states, Wg, W1, W3, W2):
    b, s, d = hidden_states.shape
    E = Wg.shape[1]
    S = b * s
    P = (E + max(0, 2 * S - E) // MB) * MB
    x = hidden_states.reshape(S, d)

    w0, w1, dest32, blk = _run_router(x, Wg)
    idx2d = dest32.reshape(-1, _SC_CHUNK)
    blk_flat = blk.reshape(-1)

    xg = _sc_scatter(x, idx2d, P)
    y = _run_moe(blk_flat, xg, W1, W3, W2)
    yg = _sc_gather(y, idx2d, 2 * S)
    out = _run_combine(yg, w0, w1)
    return out.reshape(b, s, d)
